# baseline (speedup 1.0000x reference)
import functools

import jax
import jax.numpy as jnp
from jax import lax
from jax.experimental import pallas as pl
from jax.experimental.pallas import tpu as pltpu
from jax.experimental.pallas import tpu_sc as plsc

B = 16384
D = 64
NH = 128
NV = 100000
NW = 32
RPW = B // NW
NCHUNK = 8
CH = RPW // NCHUNK
L = 16
NK = D // L


def _sc_kernel_body(tidx_hbm, midx_hbm, tt_hbm, mt_hbm, w_hbm, out_hbm,
                    idx_tv, idx_mv, et, em, wv, ov, *sems):
    wid = lax.axis_index("s") * 2 + lax.axis_index("c")
    base = wid * RPW
    pltpu.sync_copy(tidx_hbm.at[pl.ds(base, RPW)], idx_tv.at[pl.ds(0, RPW)])
    pltpu.sync_copy(midx_hbm.at[pl.ds(base, RPW)], idx_mv.at[pl.ds(0, RPW)])
    pltpu.sync_copy(w_hbm, wv)

    def gather(c, slot, sem_pair):
        cp_t = pltpu.async_copy(tt_hbm.at[idx_tv.at[pl.ds(c * CH, CH)]],
                                et.at[slot], sem_pair[0])
        cp_m = pltpu.async_copy(mt_hbm.at[idx_mv.at[pl.ds(c * CH, CH)]],
                                em.at[slot], sem_pair[1])
        return cp_t, cp_m

    copies = [None, None]
    copies[0] = gather(0, 0, sems[0:2])
    copies[1] = gather(1, 1, sems[2:4])

    lane = lax.iota(jnp.int32, 16)
    last_lane = lane == (L - 1)

    wvec = [[wv[pl.ds((tj * D) + k * L, L)] for k in range(NK)]
            for tj in range(9)]

    def row_mac(slot, r, rglob):
        e = [et[slot, r, pl.ds(k * L, L)] for k in range(NK)]
        m = [em[slot, r, pl.ds(k * L, L)] for k in range(NK)]
        p = [e[k] * m[k] for k in range(NK)]
        for j in range(3):
            s = e[0] * wvec[j][0]
            for k in range(1, NK):
                s = s + e[k] * wvec[j][k]
            for k in range(NK):
                s = s + m[k] * wvec[3 + j][k]
            for k in range(NK):
                s = s + p[k] * wvec[6 + j][k]
            tot = lax.cumsum(s, axis=0)
            plsc.store_scatter(ov, [jnp.full((L,), 1, jnp.int32)
                                    * (j * RPW + rglob)],
                               tot, mask=last_lane)

    for c in range(NCHUNK):
        slot = c % 2
        copies[slot][0].wait()
        copies[slot][1].wait()

        def pair_body(g, carry):
            for u in range(4):
                row_mac(slot, g * 4 + u, c * CH + g * 4 + u)
            return carry

        lax.fori_loop(0, CH // 4, pair_body, 0)
        if c + 2 < NCHUNK:
            copies[slot] = gather(c + 2, slot, sems[4 + 2 * slot:6 + 2 * slot])

    for j in range(3):
        pltpu.sync_copy(ov.at[pl.ds(j * RPW, RPW)],
                        out_hbm.at[pl.ds(j * B + base, RPW)])


@jax.jit
def _sc_part(tidx, midx, tt2, mt2, w_flat):
    mesh = plsc.VectorSubcoreMesh(core_axis_name="c", subcore_axis_name="s")
    scratch = [
        pltpu.VMEM((RPW,), jnp.int32),
        pltpu.VMEM((RPW,), jnp.int32),
        pltpu.VMEM((2, CH, D), jnp.float32),
        pltpu.VMEM((2, CH, D), jnp.float32),
        pltpu.VMEM((9 * D,), jnp.float32),
        pltpu.VMEM((3 * RPW,), jnp.float32),
    ] + [pltpu.SemaphoreType.DMA] * 8
    return pl.kernel(
        _sc_kernel_body,
        mesh=mesh,
        out_type=jax.ShapeDtypeStruct((3 * B,), jnp.float32),
        scratch_types=scratch,
        compiler_params=pltpu.CompilerParams(
            needs_layout_passes=False, use_tc_tiling_on_sc=False),
    )(tidx, midx, tt2, mt2, w_flat)


def _tc_body(h_ref, w_ref, b_ref, o_ref):
    o_ref[...] = lax.dot_general(
        w_ref[...], h_ref[...], (((1,), (1,)), ((), ())),
        preferred_element_type=jnp.float32) + b_ref[...]


@jax.jit
def _tc_part(h, w3, b3):
    blk = 2048
    return pl.pallas_call(
        _tc_body,
        grid=(B // blk,),
        in_specs=[
            pl.BlockSpec((blk, NH), lambda i: (i, 0)),
            pl.BlockSpec((3, NH), lambda i: (0, 0)),
            pl.BlockSpec((3, 1), lambda i: (0, 0)),
        ],
        out_specs=pl.BlockSpec((3, blk), lambda i: (0, i)),
        out_shape=jax.ShapeDtypeStruct((3, B), jnp.float32),
    )(h, w3, b3)


def kernel(h, teacher_idx, materia_idx, teacher_table, materia_table, W, b):
    tidx = teacher_idx.astype(jnp.int32)
    midx = materia_idx.astype(jnp.int32)
    w_emb = W[NH:].reshape(3, D, 3)
    w_cols = jnp.transpose(w_emb, (0, 2, 1)).reshape(9 * D)
    sc = _sc_part(tidx, midx, teacher_table, materia_table, w_cols)
    tc = _tc_part(h, W[:NH].T, b.reshape(3, 1))
    return (tc + sc.reshape(3, B)).T


# speedup vs baseline: 1.0024x; 1.0024x over previous
import jax
import jax.numpy as jnp
from jax import lax
from jax.experimental import pallas as pl
from jax.experimental.pallas import tpu as pltpu
from jax.experimental.pallas import tpu_sc as plsc

B = 16384
D = 64
NH = 128
NV = 100000
NW = 32
RPW = B // NW
NCHUNK = 4
CH = RPW // NCHUNK
L = 16
NK = D // L


def _sc_kernel_body(tidx_hbm, midx_hbm, tt_hbm, mt_hbm, w_hbm, out_hbm,
                    idx_tv, idx_mv, et, em, wv, ov, *sems):
    wid = lax.axis_index("s") * 2 + lax.axis_index("c")
    base = wid * RPW
    pltpu.sync_copy(tidx_hbm.at[pl.ds(base, RPW)], idx_tv.at[pl.ds(0, RPW)])
    pltpu.sync_copy(midx_hbm.at[pl.ds(base, RPW)], idx_mv.at[pl.ds(0, RPW)])
    pltpu.sync_copy(w_hbm, wv)

    def gather(c, slot, sem_pair):
        cp_t = pltpu.async_copy(tt_hbm.at[idx_tv.at[pl.ds(c * CH, CH)]],
                                et.at[slot], sem_pair[0])
        cp_m = pltpu.async_copy(mt_hbm.at[idx_mv.at[pl.ds(c * CH, CH)]],
                                em.at[slot], sem_pair[1])
        return cp_t, cp_m

    copies = [None, None]
    copies[0] = gather(0, 0, sems[0:2])
    copies[1] = gather(1, 1, sems[2:4])

    lane = lax.iota(jnp.int32, 16)
    last_lane = lane == (L - 1)

    wvec = [[wv[pl.ds((tj * D) + k * L, L)] for k in range(NK)]
            for tj in range(9)]

    def row_mac(slot, r, rglob):
        e = [et[slot, r, pl.ds(k * L, L)] for k in range(NK)]
        m = [em[slot, r, pl.ds(k * L, L)] for k in range(NK)]
        p = [e[k] * m[k] for k in range(NK)]
        for j in range(3):
            s = e[0] * wvec[j][0]
            for k in range(1, NK):
                s = s + e[k] * wvec[j][k]
            for k in range(NK):
                s = s + m[k] * wvec[3 + j][k]
            for k in range(NK):
                s = s + p[k] * wvec[6 + j][k]
            tot = lax.cumsum(s, axis=0)
            plsc.store_scatter(ov, [jnp.full((L,), 1, jnp.int32)
                                    * (j * RPW + rglob)],
                               tot, mask=last_lane)

    for c in range(NCHUNK):
        slot = c % 2
        copies[slot][0].wait()
        copies[slot][1].wait()

        def pair_body(g, carry):
            for u in range(4):
                row_mac(slot, g * 4 + u, c * CH + g * 4 + u)
            return carry

        lax.fori_loop(0, CH // 4, pair_body, 0)
        if c + 2 < NCHUNK:
            copies[slot] = gather(c + 2, slot, sems[4 + 2 * slot:6 + 2 * slot])

    for j in range(3):
        pltpu.sync_copy(ov.at[pl.ds(j * RPW, RPW)],
                        out_hbm.at[pl.ds(j * B + base, RPW)])


@jax.jit
def _sc_part(tidx, midx, tt2, mt2, w_flat):
    mesh = plsc.VectorSubcoreMesh(core_axis_name="c", subcore_axis_name="s")
    scratch = [
        pltpu.VMEM((RPW,), jnp.int32),
        pltpu.VMEM((RPW,), jnp.int32),
        pltpu.VMEM((2, CH, D), jnp.float32),
        pltpu.VMEM((2, CH, D), jnp.float32),
        pltpu.VMEM((9 * D,), jnp.float32),
        pltpu.VMEM((3 * RPW,), jnp.float32),
    ] + [pltpu.SemaphoreType.DMA] * 8
    return pl.kernel(
        _sc_kernel_body,
        mesh=mesh,
        out_type=jax.ShapeDtypeStruct((3 * B,), jnp.float32),
        scratch_types=scratch,
        compiler_params=pltpu.CompilerParams(
            needs_layout_passes=False, use_tc_tiling_on_sc=False),
    )(tidx, midx, tt2, mt2, w_flat)


def _tc_body(h_ref, w_ref, b_ref, o_ref):
    o_ref[...] = lax.dot_general(
        w_ref[...], h_ref[...], (((1,), (1,)), ((), ())),
        preferred_element_type=jnp.float32) + b_ref[...]


@jax.jit
def _tc_part(h, w3, b3):
    blk = 2048
    return pl.pallas_call(
        _tc_body,
        grid=(B // blk,),
        in_specs=[
            pl.BlockSpec((blk, NH), lambda i: (i, 0)),
            pl.BlockSpec((3, NH), lambda i: (0, 0)),
            pl.BlockSpec((3, 1), lambda i: (0, 0)),
        ],
        out_specs=pl.BlockSpec((3, blk), lambda i: (0, i)),
        out_shape=jax.ShapeDtypeStruct((3, B), jnp.float32),
    )(h, w3, b3)


def kernel(h, teacher_idx, materia_idx, teacher_table, materia_table, W, b):
    tidx = teacher_idx.astype(jnp.int32)
    midx = materia_idx.astype(jnp.int32)
    w_emb = W[NH:].reshape(3, D, 3)
    w_cols = jnp.transpose(w_emb, (0, 2, 1)).reshape(9 * D)
    sc = _sc_part(tidx, midx, teacher_table, materia_table, w_cols)
    tc = _tc_part(h, W[:NH].T, b.reshape(3, 1))
    return (tc + sc.reshape(3, B)).T
